# baseline (speedup 1.0000x reference)
import functools

import jax
import jax.numpy as jnp
from jax import lax
from jax.experimental import pallas as pl
from jax.experimental.pallas import tpu as pltpu
from jax.experimental.pallas import tpu_sc as plsc

DIM = 64
NUM_CORES = 2
NUM_SUBCORES = 16
NUM_WORKERS = NUM_CORES * NUM_SUBCORES
BB = 128


def _embed_sc(xT, W, b, s):
    assert b == NUM_WORKERS * BB and DIM == 64
    mesh = plsc.VectorSubcoreMesh(core_axis_name="c", subcore_axis_name="s")

    @functools.partial(
        pl.kernel,
        mesh=mesh,
        compiler_params=pltpu.CompilerParams(
            use_tc_tiling_on_sc=False, needs_layout_passes=False),
        out_type=jax.ShapeDtypeStruct((s, DIM // 8, b // BB, 8, BB), jnp.float32),
        scratch_types=[
            pltpu.VMEM((s, BB), jnp.int32),
            [pltpu.VMEM((BB, DIM), jnp.float32)] * 2,
            [pltpu.VMEM((DIM // 8, 8, BB), jnp.float32)] * 2,
            [pltpu.SemaphoreType.DMA] * 2,
            [pltpu.SemaphoreType.DMA] * 2,
        ],
    )
    def k(W_hbm, xT_hbm, out_hbm, xv, rows, rT, gsem, ssem):
        wid = lax.axis_index("s") * NUM_CORES + lax.axis_index("c")
        pltpu.sync_copy(xT_hbm.at[:, pl.ds(wid * BB, BB)], xv)

        lane = jnp.arange(16, dtype=jnp.int32)
        idx_b = [lane + (g * 16) for g in range(8)]

        def gather_copy(j, buf):
            return pltpu.make_async_copy(W_hbm.at[xv.at[j]], rows[buf], gsem[buf])

        def store_copy(j, buf):
            return pltpu.make_async_copy(rT[buf], out_hbm.at[j, :, wid], ssem[buf])

        def transpose_block(buf):
            @pl.loop(0, DIM, unroll=4)
            def _(d):
                idx_d = jnp.full((16,), d, dtype=jnp.int32)
                dblk = lax.div(d, 8)
                din = lax.rem(d, 8)
                for g in range(8):
                    v = plsc.load_gather(rows[buf], [idx_b[g], idx_d])
                    rT[buf][dblk, din, pl.ds(g * 16, 16)] = v

        gather_copy(0, 0).start()
        gather_copy(1, 1).start()
        for j in (0, 1):
            gather_copy(j, j).wait()
            transpose_block(j)
            store_copy(j, j).start()
            gather_copy(j + 2, j).start()

        @pl.loop(2, s - 2, step=2)
        def _(i):
            for buf in range(2):
                j = i + buf
                store_copy(j, buf).wait()
                gather_copy(j, buf).wait()
                transpose_block(buf)
                store_copy(j, buf).start()
                gather_copy(j + 2, buf).start()

        for jj in (s - 2, s - 1):
            buf = jj % 2
            store_copy(jj, buf).wait()
            gather_copy(jj, buf).wait()
            transpose_block(buf)
            store_copy(jj, buf).start()
        for buf in range(2):
            store_copy(s - 2 + buf, buf).wait()

    return k(W, xT)


def kernel(x, W):
    b, s = x.shape
    xT = x.T.astype(jnp.int32)
    out5 = _embed_sc(xT, W.astype(jnp.float32), b, s)
    return out5.transpose(2, 4, 0, 1, 3).reshape(b, s, DIM)


# speedup vs baseline: 1.2281x; 1.2281x over previous
import functools

import jax
import jax.numpy as jnp
from jax import lax
from jax.experimental import pallas as pl
from jax.experimental.pallas import tpu as pltpu
from jax.experimental.pallas import tpu_sc as plsc

DIM = 64
NUM_CORES = 2
NUM_SUBCORES = 16
NUM_WORKERS = NUM_CORES * NUM_SUBCORES
BB = 128


def _embed_sc(xT, W, b, s):
    assert b == NUM_WORKERS * BB and DIM == 64
    mesh = plsc.VectorSubcoreMesh(core_axis_name="c", subcore_axis_name="s")

    @functools.partial(
        pl.kernel,
        mesh=mesh,
        compiler_params=pltpu.CompilerParams(
            use_tc_tiling_on_sc=False, needs_layout_passes=False),
        out_type=jax.ShapeDtypeStruct((s, DIM // 8, b // BB, 8, BB), jnp.float32),
        scratch_types=[
            pltpu.VMEM((s, BB), jnp.int32),
            [pltpu.VMEM((BB, DIM), jnp.float32)] * 2,
            [pltpu.VMEM((DIM // 8, 8, BB), jnp.float32)] * 2,
            [pltpu.SemaphoreType.DMA] * 2,
            [pltpu.SemaphoreType.DMA] * 2,
        ],
    )
    def k(W_hbm, xT_hbm, out_hbm, xv, rows, rT, gsem, ssem):
        wid = lax.axis_index("s") * NUM_CORES + lax.axis_index("c")
        pltpu.sync_copy(xT_hbm.at[:, pl.ds(wid * BB, BB)], xv)

        lane = jnp.arange(16, dtype=jnp.int32)
        idx_b = [lane + (g * 16) for g in range(8)]

        def gather_copy(j, buf):
            return pltpu.make_async_copy(W_hbm.at[xv.at[j]], rows[buf], gsem[buf])

        def store_copy(j, buf):
            return pltpu.make_async_copy(rT[buf], out_hbm.at[j, :, wid], ssem[buf])

        def transpose_block(buf):
            @pl.loop(0, DIM, unroll=2)
            def _(d):
                idx_d = jnp.full((16,), d, dtype=jnp.int32)
                dblk = lax.div(d, 8)
                din = lax.rem(d, 8)
                vs = [plsc.load_gather(rows[buf], [idx_b[g], idx_d])
                      for g in range(8)]
                for g in range(8):
                    rT[buf][dblk, din, pl.ds(g * 16, 16)] = vs[g]

        gather_copy(0, 0).start()
        gather_copy(1, 1).start()
        for j in (0, 1):
            gather_copy(j, j).wait()
            transpose_block(j)
            store_copy(j, j).start()
            gather_copy(j + 2, j).start()

        @pl.loop(2, s - 2, step=2)
        def _(i):
            for buf in range(2):
                j = i + buf
                store_copy(j, buf).wait()
                gather_copy(j, buf).wait()
                transpose_block(buf)
                store_copy(j, buf).start()
                gather_copy(j + 2, buf).start()

        for jj in (s - 2, s - 1):
            buf = jj % 2
            store_copy(jj, buf).wait()
            gather_copy(jj, buf).wait()
            transpose_block(buf)
            store_copy(jj, buf).start()
        for buf in range(2):
            store_copy(s - 2 + buf, buf).wait()

    return k(W, xT)


def kernel(x, W):
    b, s = x.shape
    xT = x.T.astype(jnp.int32)
    out5 = _embed_sc(xT, W.astype(jnp.float32), b, s)
    return out5.transpose(2, 4, 0, 1, 3).reshape(b, s, DIM)
